# baseline (speedup 1.0000x reference)
import functools
import math

import jax
import jax.numpy as jnp
from jax import lax
from jax.experimental import pallas as pl
from jax.experimental.pallas import tpu as pltpu
from jax.experimental.pallas import tpu_sc as plsc

_NC = 2
_NS = 16
_NW = _NC * _NS
_LANES = 16
_IDX_W = 128
_LN10 = math.log(10.0)


@functools.partial(jax.jit, static_argnames=("rows_per_w",))
def _run(mu, log_sigma, noise2d, ids2d, rows_per_w):
    mesh = plsc.VectorSubcoreMesh(core_axis_name="c", subcore_axis_name="s")

    @functools.partial(
        pl.kernel,
        out_type=jax.ShapeDtypeStruct(noise2d.shape, jnp.float32),
        mesh=mesh,
        scratch_types=[
            pltpu.VMEM((rows_per_w, _IDX_W), jnp.int32),
            pltpu.VMEM((rows_per_w, _IDX_W), jnp.float32),
            pltpu.VMEM((rows_per_w, _IDX_W), jnp.float32),
            pltpu.VMEM((rows_per_w, _IDX_W), jnp.float32),
            pltpu.VMEM((rows_per_w, _IDX_W), jnp.float32),
            pltpu.SemaphoreType.DMA,
            pltpu.SemaphoreType.DMA,
            [pltpu.SemaphoreType.DMA] * rows_per_w,
        ],
    )
    def body(mu_hbm, ls_hbm, noise_hbm, ids_hbm, out_hbm,
             idx_v, mu_v, ls_v, noise_v, out_v,
             sem_idx, sem_noise, sems_g):
        wid = lax.axis_index("s") * _NC + lax.axis_index("c")
        base = wid * rows_per_w
        ids_cp = pltpu.async_copy(ids_hbm.at[pl.ds(base, rows_per_w)], idx_v, sem_idx)
        noise_cp = pltpu.async_copy(noise_hbm.at[pl.ds(base, rows_per_w)], noise_v, sem_noise)
        ids_cp.wait()
        copies = []
        for j in range(rows_per_w):
            copies.append((
                pltpu.async_copy(mu_hbm.at[idx_v.at[j]], mu_v.at[j], sems_g[j]),
                pltpu.async_copy(ls_hbm.at[idx_v.at[j]], ls_v.at[j], sems_g[j]),
            ))
        noise_cp.wait()
        for j in range(rows_per_w):
            copies[j][0].wait()
            copies[j][1].wait()

            def compute_block(i, _, j=j):
                sl = pl.ds(i * _LANES, _LANES)
                m = mu_v[j, sl]
                g = ls_v[j, sl]
                nz = noise_v[j, sl]
                out_v[j, sl] = jnp.exp((m + jnp.exp(g) * nz) * _LN10)
                return _

            lax.fori_loop(0, _IDX_W // _LANES, compute_block, 0)
        pltpu.sync_copy(out_v, out_hbm.at[pl.ds(base, rows_per_w)])

    return body(mu, log_sigma, noise2d, ids2d)


def kernel(mu, log_sigma, noise, family_ids, batch_size):
    b = noise.shape[0]
    rows = b // _IDX_W
    rows_per_w = rows // _NW
    noise2d = noise.reshape(rows, _IDX_W)
    ids2d = family_ids.reshape(rows, _IDX_W)
    out = _run(mu, log_sigma, noise2d, ids2d, rows_per_w)
    return out.reshape(b)
